# initial kernel scaffold (unmeasured)
import jax
import jax.numpy as jnp
from jax import lax
from jax.experimental import pallas as pl
from jax.experimental.pallas import tpu as pltpu

N_DEV = 4
M_BLK = 1024
K = 4096
N = 8192
N_GRID = 4
N_BLK = N // N_GRID


def kernel(x, w_mat, scale_x, scale_w):
    m_glob, k_sh = x.shape
    assert (m_glob, k_sh) == (K, K // N_DEV)

    def body(x_ref, w_ref, sx_ref, sw_ref, out_ref,
             xg_ref, send_sems, recv_sems):
        pid = pl.program_id(0)
        my = lax.axis_index("i")

        @pl.when(pid == 0)
        def _comm():
            barrier_sem = pltpu.get_barrier_semaphore()
            for d in range(1, N_DEV):
                peer = lax.rem(my + d, N_DEV)
                pl.semaphore_signal(
                    barrier_sem, inc=1,
                    device_id=(peer,), device_id_type=pl.DeviceIdType.MESH,
                )
            pl.semaphore_wait(barrier_sem, N_DEV - 1)

            xg_ref[:, pl.ds(my * k_sh, k_sh)] = x_ref[pl.ds(my * M_BLK, M_BLK), :]

            rdmas = []
            for d in range(1, N_DEV):
                tgt = lax.rem(my + d, N_DEV)
                rdma = pltpu.make_async_remote_copy(
                    src_ref=x_ref.at[pl.ds(tgt * M_BLK, M_BLK), :],
                    dst_ref=xg_ref.at[:, pl.ds(my * k_sh, k_sh)],
                    send_sem=send_sems.at[d - 1],
                    recv_sem=recv_sems.at[my],
                    device_id=(tgt,),
                    device_id_type=pl.DeviceIdType.MESH,
                )
                rdma.start()
                rdmas.append(rdma)

            for rdma in rdmas:
                rdma.wait_send()

            for d in range(1, N_DEV):
                src = lax.rem(my + d, N_DEV)
                recv = pltpu.make_async_remote_copy(
                    src_ref=x_ref.at[pl.ds(0, M_BLK), :],
                    dst_ref=xg_ref.at[:, pl.ds(src * k_sh, k_sh)],
                    send_sem=send_sems.at[0],
                    recv_sem=recv_sems.at[src],
                    device_id=(src,),
                    device_id_type=pl.DeviceIdType.MESH,
                )
                recv.wait_recv()

        acc = jnp.dot(xg_ref[:, :], w_ref[:, :],
                      preferred_element_type=jnp.int32)
        scale = sx_ref[0] * sw_ref[0]
        out_ref[:, :] = acc.astype(jnp.float32) * scale

    grid = (N_GRID,)
    return pl.pallas_call(
        body,
        grid=grid,
        in_specs=[
            pl.BlockSpec((K, k_sh), lambda i: (0, 0)),
            pl.BlockSpec((K, N_BLK), lambda i: (0, i)),
            pl.BlockSpec(memory_space=pltpu.SMEM),
            pl.BlockSpec(memory_space=pltpu.SMEM),
        ],
        out_specs=pl.BlockSpec((M_BLK, N_BLK), lambda i: (0, i)),
        out_shape=jax.ShapeDtypeStruct((M_BLK, N), jnp.float32),
        scratch_shapes=[
            pltpu.VMEM((M_BLK, K), jnp.int8),
            pltpu.SemaphoreType.DMA((N_DEV - 1,)),
            pltpu.SemaphoreType.DMA((N_DEV,)),
        ],
        compiler_params=pltpu.CompilerParams(collective_id=0),
    )(x, w_mat, scale_x, scale_w)


# baseline (device time: 128613 ns/iter reference)
import jax
import jax.numpy as jnp
from jax import lax
from jax.experimental import pallas as pl
from jax.experimental.pallas import tpu as pltpu

N_DEV = 4
M_BLK = 1024
K = 4096
N = 8192
N_GRID = 8
N_BLK = N // N_GRID


def kernel(x, w_mat, scale_x, scale_w):
    m_glob, k_sh = x.shape
    assert (m_glob, k_sh) == (K, K // N_DEV)

    def body(x_ref, w_ref, sx_ref, sw_ref, out_ref,
             xg_ref, send_sems, recv_sems):
        pid = pl.program_id(0)
        my = lax.axis_index("i")

        @pl.when(pid == 0)
        def _comm():
            barrier_sem = pltpu.get_barrier_semaphore()
            for d in range(1, N_DEV):
                peer = lax.rem(my + d, N_DEV)
                pl.semaphore_signal(
                    barrier_sem, inc=1,
                    device_id=(peer,), device_id_type=pl.DeviceIdType.MESH,
                )
            pl.semaphore_wait(barrier_sem, N_DEV - 1)

            xg_ref[:, pl.ds(my * k_sh, k_sh)] = x_ref[pl.ds(my * M_BLK, M_BLK), :]

            rdmas = []
            for d in range(1, N_DEV):
                tgt = lax.rem(my + d, N_DEV)
                rdma = pltpu.make_async_remote_copy(
                    src_ref=x_ref.at[pl.ds(tgt * M_BLK, M_BLK), :],
                    dst_ref=xg_ref.at[:, pl.ds(my * k_sh, k_sh)],
                    send_sem=send_sems.at[d - 1],
                    recv_sem=recv_sems.at[my],
                    device_id=(tgt,),
                    device_id_type=pl.DeviceIdType.MESH,
                )
                rdma.start()
                rdmas.append(rdma)

            for rdma in rdmas:
                rdma.wait_send()

            for d in range(1, N_DEV):
                src = lax.rem(my + d, N_DEV)
                recv = pltpu.make_async_remote_copy(
                    src_ref=x_ref.at[pl.ds(0, M_BLK), :],
                    dst_ref=xg_ref.at[:, pl.ds(src * k_sh, k_sh)],
                    send_sem=send_sems.at[0],
                    recv_sem=recv_sems.at[src],
                    device_id=(src,),
                    device_id_type=pl.DeviceIdType.MESH,
                )
                recv.wait_recv()

        acc = jnp.dot(xg_ref[:, :], w_ref[:, :],
                      preferred_element_type=jnp.int32)
        scale = sx_ref[0] * sw_ref[0]
        out_ref[:, :] = acc.astype(jnp.float32) * scale

    grid = (N_GRID,)
    return pl.pallas_call(
        body,
        grid=grid,
        in_specs=[
            pl.BlockSpec((K, k_sh), lambda i: (0, 0)),
            pl.BlockSpec((K, N_BLK), lambda i: (0, i)),
            pl.BlockSpec(memory_space=pltpu.SMEM),
            pl.BlockSpec(memory_space=pltpu.SMEM),
        ],
        out_specs=pl.BlockSpec((M_BLK, N_BLK), lambda i: (0, i)),
        out_shape=jax.ShapeDtypeStruct((M_BLK, N), jnp.float32),
        scratch_shapes=[
            pltpu.VMEM((M_BLK, K), jnp.int8),
            pltpu.SemaphoreType.DMA((N_DEV - 1,)),
            pltpu.SemaphoreType.DMA((N_DEV,)),
        ],
        compiler_params=pltpu.CompilerParams(
            collective_id=0,
            vmem_limit_bytes=100 * 1024 * 1024,
        ),
    )(x, w_mat, scale_x, scale_w)
